# baseline (speedup 1.0000x reference)
import functools

import jax
import jax.numpy as jnp
from jax import lax
from jax.experimental import pallas as pl
from jax.experimental.pallas import tpu as pltpu
from jax.experimental.pallas import tpu_sc as plsc

NC = 2
NS = 16
L = 16
NW = NC * NS

EMB = 64
CHUNK = 128


def _make_sc_kernel(batch: int):
    bpw = batch // NW
    nchunk = bpw // CHUNK
    nblk = bpw // L

    mesh = plsc.VectorSubcoreMesh(core_axis_name="c", subcore_axis_name="s")

    @functools.partial(
        pl.kernel,
        mesh=mesh,
        compiler_params=pltpu.CompilerParams(
            needs_layout_passes=False, use_tc_tiling_on_sc=False),
        out_type=jax.ShapeDtypeStruct((batch,), jnp.float32),
        scratch_types=[
            pltpu.VMEM((nchunk, CHUNK), jnp.int32),
            pltpu.VMEM((nchunk, CHUNK), jnp.int32),
            pltpu.VMEM((bpw, EMB), jnp.float32),
            pltpu.VMEM((bpw, EMB), jnp.float32),
            pltpu.VMEM((bpw,), jnp.float32),
            pltpu.VMEM((bpw,), jnp.float32),
            pltpu.VMEM((L,), jnp.float32),
            pltpu.VMEM((bpw,), jnp.float32),
            pltpu.SemaphoreType.DMA,
        ],
    )
    def sc_kernel(uid_hbm, iid_hbm, uemb_hbm, ubias_hbm, iemb_hbm, ibias_hbm,
                  mean_hbm, out_hbm,
                  uid_v, iid_v, urows_v, irows_v, ubias_v, ibias_v, mean_v,
                  out_v, sem):
        wid = lax.axis_index("s") * NC + lax.axis_index("c")

        pltpu.sync_copy(uid_hbm.at[wid], uid_v)
        pltpu.sync_copy(iid_hbm.at[wid], iid_v)
        pltpu.sync_copy(mean_hbm, mean_v)

        copies = []
        for j in range(nchunk):
            dst = pl.ds(j * CHUNK, CHUNK)
            copies.append(pltpu.async_copy(uemb_hbm.at[uid_v.at[j]], urows_v.at[dst], sem))
            copies.append(pltpu.async_copy(iemb_hbm.at[iid_v.at[j]], irows_v.at[dst], sem))
            copies.append(pltpu.async_copy(ubias_hbm.at[uid_v.at[j]], ubias_v.at[dst], sem))
            copies.append(pltpu.async_copy(ibias_hbm.at[iid_v.at[j]], ibias_v.at[dst], sem))
        for c in copies:
            c.wait()

        mean_vec = mean_v[...]
        lanes = lax.iota(jnp.int32, L)

        def blk_body(blk, carry):
            base = blk * L
            row = base + lanes
            acc = ubias_v[pl.ds(base, L)] + ibias_v[pl.ds(base, L)] + mean_vec
            for d in range(EMB):
                col = jnp.full((L,), d, jnp.int32)
                u = plsc.load_gather(urows_v, [row, col])
                iv = plsc.load_gather(irows_v, [row, col])
                acc = acc + u * iv
            out_v[pl.ds(base, L)] = acc
            return carry

        lax.fori_loop(0, nblk, blk_body, 0)

        pltpu.sync_copy(out_v, out_hbm.at[pl.ds(wid * bpw, bpw)])

    return sc_kernel


def kernel(u_id, i_id, user_emb, user_bias, item_emb, item_bias,
           user_visual_emb, mean):
    batch = u_id.shape[0]
    uid3 = u_id.reshape(NW, batch // NW // CHUNK, CHUNK)
    iid3 = i_id.reshape(NW, batch // NW // CHUNK, CHUNK)
    ubias_flat = user_bias.reshape(-1)
    ibias_flat = item_bias.reshape(-1)
    mean_l = jnp.broadcast_to(mean, (L,))
    sc = _make_sc_kernel(batch)
    return sc(uid3, iid3, user_emb, ubias_flat, item_emb, ibias_flat, mean_l)
